# baseline (speedup 1.0000x reference)
import functools

import jax
import jax.numpy as jnp
from jax import lax
from jax.experimental import pallas as pl
from jax.experimental.pallas import tpu as pltpu
from jax.experimental.pallas import tpu_sc as plsc

_VOCAB = 8
_EMB = 32
_CTX = 2


def _sc_body(x_hbm, emb_hbm, w_hbm, b_hbm, out_hbm,
             x_s, emb_s, w_s, b_s, out_s, sem):
    cp_x = pltpu.async_copy(x_hbm, x_s, sem)
    cp_e = pltpu.async_copy(emb_hbm, emb_s, sem)
    cp_w = pltpu.async_copy(w_hbm, w_s, sem)
    cp_b = pltpu.async_copy(b_hbm, b_s, sem)
    cp_x.wait()
    cp_e.wait()
    cp_w.wait()
    cp_b.wait()

    base0 = x_s[0] * _EMB
    base1 = x_s[1] * _EMB

    def step(j, accs):
        e0 = emb_s[base0 + j]
        e1 = emb_s[base1 + j]
        return tuple(
            accs[v] + w_s[v * _EMB * _CTX + j] * e0
                    + w_s[v * _EMB * _CTX + _EMB + j] * e1
            for v in range(_VOCAB)
        )

    accs = lax.fori_loop(0, _EMB, step,
                         tuple(b_s[v] for v in range(_VOCAB)))
    for v in range(_VOCAB):
        out_s[v] = accs[v]
    pltpu.sync_copy(out_s, out_hbm.at[0])


_sc_call = functools.partial(
    pl.kernel,
    mesh=plsc.ScalarSubcoreMesh(axis_name="c", num_cores=1),
    out_type=jax.ShapeDtypeStruct((1, _VOCAB), jnp.float32),
    scratch_types=[
        pltpu.SMEM((_CTX,), jnp.int32),
        pltpu.SMEM((_VOCAB * _EMB,), jnp.float32),
        pltpu.SMEM((_VOCAB * _EMB * _CTX,), jnp.float32),
        pltpu.SMEM((_VOCAB,), jnp.float32),
        pltpu.SMEM((_VOCAB,), jnp.float32),
        pltpu.SemaphoreType.DMA,
    ],
    compiler_params=pltpu.CompilerParams(needs_layout_passes=False),
)(_sc_body)


def kernel(x, emb, W, b):
    return _sc_call(x.astype(jnp.int32), emb.reshape(-1), W.reshape(-1), b)
